# baseline (speedup 1.0000x reference)
import functools

import jax
import jax.numpy as jnp
from jax import lax
from jax.experimental import pallas as pl
from jax.experimental.pallas import tpu as pltpu
from jax.experimental.pallas import tpu_sc as plsc

_F = 26
_D = 16
_B = 16384
_V = 1040000
_C = 256
_UPF = 1024 // _C
_HC = 8 * _C
_DPLANE = (_V // 128) * 1024
_NR = 314
_SPLANE = _NR * 1024
_SSLOT = 2 * _SPLANE
_SHARE = _SPLANE // 16
_FPC = _F // 2

_mesh = plsc.VectorSubcoreMesh(core_axis_name="c", subcore_axis_name="s")

_scratch = [pltpu.VMEM_SHARED((2 * _SSLOT,), jnp.float32),
            pltpu.SemaphoreType.DMA,
            pltpu.VMEM((2 * 1024,), jnp.int32),
            pltpu.SemaphoreType.DMA]
for _ in range(_UPF):
    _scratch += [
        pltpu.VMEM((_HC,), jnp.int32),
        pltpu.VMEM((_HC,), jnp.int32),
        pltpu.VMEM((_HC,), jnp.float32),
        pltpu.VMEM((_HC,), jnp.float32),
        pltpu.SemaphoreType.DMA,
        pltpu.SemaphoreType.DMA,
        pltpu.SemaphoreType.DMA,
    ]


@functools.partial(
    pl.kernel,
    mesh=_mesh,
    out_type=jax.ShapeDtypeStruct((_F * _D * _B,), jnp.float32),
    scratch_types=_scratch,
    compiler_params=pltpu.CompilerParams(use_tc_tiling_on_sc=True),
)
def _emb_lookup(w1d_hbm, xt_hbm, out_hbm, spmem, asem, xbuf, xsem, *bufs):
    core = lax.axis_index("c")
    tid = lax.axis_index("s")
    sets = [bufs[7 * i: 7 * i + 7] for i in range(_UPF)]

    def field_of(j):
        return 2 * j + core

    def rs_of(f):
        r0 = (40000 * f) >> 7
        return jnp.minimum(r0, (_V // 128) - _NR)

    def fire_phase_a(j, slot):
        f = field_of(j)
        rs = rs_of(f)
        for dpl in range(2):
            src = dpl * _DPLANE + rs * 1024 + tid * _SHARE
            dstw = slot * _SSLOT + dpl * _SPLANE + tid * _SHARE
            pltpu.async_copy(w1d_hbm.at[pl.ds(src, _SHARE)],
                             spmem.at[pl.ds(dstw, _SHARE)], asem)

    def wait_phase_a():
        for _ in range(2):
            pltpu.make_async_copy(w1d_hbm.at[pl.ds(0, _SHARE)],
                                  spmem.at[pl.ds(0, _SHARE)], asem).wait()

    def fire_x(j, slot):
        f = field_of(j)
        pltpu.async_copy(xt_hbm.at[f, pl.ds(tid * 1024, 1024)],
                         xbuf.at[pl.ds(slot * 1024, 1024)], xsem)

    def wait_x():
        pltpu.make_async_copy(xt_hbm.at[0, pl.ds(0, 1024)],
                              xbuf.at[pl.ds(0, 1024)], xsem).wait()

    def stage(u, s, f, slot, addc, j):
        idx0, idx1, dst0, dst1, g0, g1, ws = s

        @pl.when(j > 0)
        def _():
            pltpu.make_async_copy(dst0, out_hbm.at[pl.ds(0, _HC)], ws).wait()
            pltpu.make_async_copy(dst1, out_hbm.at[pl.ds(0, _HC)], ws).wait()

        off = 40000 * f
        for g in range(_C // 16):
            r = xbuf[pl.ds(slot * 1024 + u * _C + 16 * g, 16)] + off
            base = ((r >> 7) << 10) + (r & 127) + addc
            sl = (g // 8) * 1024 + (g % 8) * 16
            for sub in range(8):
                idx0[pl.ds(sl + sub * 128, 16)] = base + (sub * 128)
                idx1[pl.ds(sl + sub * 128, 16)] = base + (_SPLANE + sub * 128)
        h = _HC // 2
        pltpu.async_copy(spmem.at[idx0.at[pl.ds(0, h)]], dst0.at[pl.ds(0, h)], g0)
        pltpu.async_copy(spmem.at[idx0.at[pl.ds(h, h)]], dst0.at[pl.ds(h, h)], g0)
        pltpu.async_copy(spmem.at[idx1.at[pl.ds(0, h)]], dst1.at[pl.ds(0, h)], g1)
        pltpu.async_copy(spmem.at[idx1.at[pl.ds(h, h)]], dst1.at[pl.ds(h, h)], g1)

    def drain(u, s, f):
        idx0, idx1, dst0, dst1, g0, g1, ws = s
        b0 = tid * 1024 + u * _C
        obase = f * (_D * _B) + (b0 // 128) * 1024
        h = _HC // 2
        for _ in range(2):
            pltpu.make_async_copy(spmem.at[idx0.at[pl.ds(0, h)]],
                                  dst0.at[pl.ds(0, h)], g0).wait()
            pltpu.make_async_copy(spmem.at[idx1.at[pl.ds(0, h)]],
                                  dst1.at[pl.ds(0, h)], g1).wait()
        pltpu.async_copy(dst0, out_hbm.at[pl.ds(obase, _HC)], ws)
        pltpu.async_copy(dst1, out_hbm.at[pl.ds(obase + 8 * _B, _HC)], ws)

    fire_phase_a(0, 0)
    fire_x(0, 0)

    def body(j, carry):
        slot = j & 1
        f = field_of(j)
        wait_phase_a()
        plsc.subcore_barrier()
        nxt = jnp.minimum(j + 1, _FPC - 1)
        fire_phase_a(nxt, 1 - slot)
        fire_x(nxt, 1 - slot)
        wait_x()
        addc = slot * _SSLOT - rs_of(f) * 1024
        for u in range(_UPF):
            stage(u, sets[u], f, slot, addc, j)
        for u in range(_UPF):
            drain(u, sets[u], f)
        return carry

    lax.fori_loop(0, _FPC, body, 0)
    wait_phase_a()
    wait_x()
    for s in sets:
        _, _, dst0, dst1, _, _, ws = s
        pltpu.make_async_copy(dst0, out_hbm.at[pl.ds(0, _HC)], ws).wait()
        pltpu.make_async_copy(dst1, out_hbm.at[pl.ds(0, _HC)], ws).wait()


def kernel(x, weight):
    wt = weight.T
    w1d = (wt.reshape(2, 8, _V // 128, 128)
             .transpose(0, 2, 1, 3)
             .reshape(-1))
    xt = x.T
    out1d = _emb_lookup(w1d, xt)
    return (out1d.reshape(_F, 2, _B // 128, 8, 128)
                 .transpose(2, 4, 0, 1, 3)
                 .reshape(_B, _F, _D))


# speedup vs baseline: 1.0030x; 1.0030x over previous
import functools

import jax
import jax.numpy as jnp
from jax import lax
from jax.experimental import pallas as pl
from jax.experimental.pallas import tpu as pltpu
from jax.experimental.pallas import tpu_sc as plsc

_F = 26
_D = 16
_B = 16384
_V = 1040000
_C = 256
_UPF = 1024 // _C
_HC = 8 * _C
_DPLANE = (_V // 128) * 1024
_NR = 314
_SPLANE = _NR * 1024
_SSLOT = 2 * _SPLANE
_SHARE = _SPLANE // 16
_FPC = _F // 2

_mesh = plsc.VectorSubcoreMesh(core_axis_name="c", subcore_axis_name="s")

_scratch = [pltpu.VMEM_SHARED((2 * _SSLOT,), jnp.float32),
            pltpu.SemaphoreType.DMA,
            pltpu.VMEM((2 * 1024,), jnp.int32),
            pltpu.SemaphoreType.DMA]
for _ in range(_UPF):
    _scratch += [
        pltpu.VMEM((_HC,), jnp.int32),
        pltpu.VMEM((_HC,), jnp.int32),
        pltpu.VMEM((_HC,), jnp.float32),
        pltpu.VMEM((_HC,), jnp.float32),
        pltpu.SemaphoreType.DMA,
        pltpu.SemaphoreType.DMA,
        pltpu.SemaphoreType.DMA,
    ]


@functools.partial(
    pl.kernel,
    mesh=_mesh,
    out_type=jax.ShapeDtypeStruct((_F * _D * _B,), jnp.float32),
    scratch_types=_scratch,
    compiler_params=pltpu.CompilerParams(use_tc_tiling_on_sc=True),
)
def _emb_lookup(w1d_hbm, xt_hbm, out_hbm, spmem, asem, xbuf, xsem, *bufs):
    core = lax.axis_index("c")
    tid = lax.axis_index("s")
    sets = [bufs[7 * i: 7 * i + 7] for i in range(_UPF)]

    def field_of(j):
        return 2 * j + core

    def rs_of(f):
        r0 = (40000 * f) >> 7
        return jnp.minimum(r0, (_V // 128) - _NR)

    def fire_phase_a(j, slot):
        f = field_of(j)
        rs = rs_of(f)
        for dpl in range(2):
            src = dpl * _DPLANE + rs * 1024 + tid * _SHARE
            dstw = slot * _SSLOT + dpl * _SPLANE + tid * _SHARE
            pltpu.async_copy(w1d_hbm.at[pl.ds(src, _SHARE)],
                             spmem.at[pl.ds(dstw, _SHARE)], asem)

    def wait_phase_a():
        for _ in range(2):
            pltpu.make_async_copy(w1d_hbm.at[pl.ds(0, _SHARE)],
                                  spmem.at[pl.ds(0, _SHARE)], asem).wait()

    def fire_x(j, slot):
        f = field_of(j)
        pltpu.async_copy(xt_hbm.at[f, pl.ds(tid * 1024, 1024)],
                         xbuf.at[pl.ds(slot * 1024, 1024)], xsem)

    def wait_x():
        pltpu.make_async_copy(xt_hbm.at[0, pl.ds(0, 1024)],
                              xbuf.at[pl.ds(0, 1024)], xsem).wait()

    def stage(u, s, f, slot, addc, j):
        idx0, idx1, dst0, dst1, g0, g1, ws = s

        @pl.when(j > 0)
        def _():
            pltpu.make_async_copy(dst0, out_hbm.at[pl.ds(0, _HC)], ws).wait()
            pltpu.make_async_copy(dst1, out_hbm.at[pl.ds(0, _HC)], ws).wait()

        off = 40000 * f
        for g in range(_C // 16):
            r = xbuf[pl.ds(slot * 1024 + u * _C + 16 * g, 16)] + off
            base = ((r >> 7) << 10) + (r & 127) + addc
            sl = (g // 8) * 1024 + (g % 8) * 16
            for sub in range(8):
                idx0[pl.ds(sl + sub * 128, 16)] = base + (sub * 128)
                idx1[pl.ds(sl + sub * 128, 16)] = base + (_SPLANE + sub * 128)
        pltpu.async_copy(spmem.at[idx0], dst0, g0)
        pltpu.async_copy(spmem.at[idx1], dst1, g1)

    def drain(u, s, f):
        idx0, idx1, dst0, dst1, g0, g1, ws = s
        b0 = tid * 1024 + u * _C
        obase = f * (_D * _B) + (b0 // 128) * 1024
        pltpu.make_async_copy(spmem.at[idx0], dst0, g0).wait()
        pltpu.make_async_copy(spmem.at[idx1], dst1, g1).wait()
        pltpu.async_copy(dst0, out_hbm.at[pl.ds(obase, _HC)], ws)
        pltpu.async_copy(dst1, out_hbm.at[pl.ds(obase + 8 * _B, _HC)], ws)

    fire_phase_a(0, 0)
    fire_x(0, 0)

    def body(j, carry):
        slot = j & 1
        f = field_of(j)
        wait_phase_a()
        plsc.subcore_barrier()
        nxt = jnp.minimum(j + 1, _FPC - 1)
        fire_phase_a(nxt, 1 - slot)
        fire_x(nxt, 1 - slot)
        wait_x()
        addc = slot * _SSLOT - rs_of(f) * 1024
        for u in range(_UPF):
            stage(u, sets[u], f, slot, addc, j)
        for u in range(_UPF):
            drain(u, sets[u], f)
        return carry

    lax.fori_loop(0, _FPC, body, 0)
    wait_phase_a()
    wait_x()
    for s in sets:
        _, _, dst0, dst1, _, _, ws = s
        pltpu.make_async_copy(dst0, out_hbm.at[pl.ds(0, _HC)], ws).wait()
        pltpu.make_async_copy(dst1, out_hbm.at[pl.ds(0, _HC)], ws).wait()


def kernel(x, weight):
    wt = weight.T
    w1d = (wt.reshape(2, 8, _V // 128, 128)
             .transpose(0, 2, 1, 3)
             .reshape(-1))
    xt = x.T
    out1d = _emb_lookup(w1d, xt)
    return (out1d.reshape(_F, 2, _B // 128, 8, 128)
                 .transpose(2, 4, 0, 1, 3)
                 .reshape(_B, _F, _D))
